# Initial kernel scaffold; baseline (speedup 1.0000x reference)
import jax
import jax.numpy as jnp
from jax.experimental import pallas as pl


def kernel(nodes_real, indices, v, unique_nodes_list, num_sample, W):
    raise NotImplementedError("write your pallas kernel here")


# speedup vs baseline: 1.5451x; 1.5451x over previous
import dataclasses
import functools

import jax
import jax.numpy as jnp
from jax import lax
from jax.experimental import pallas as pl
from jax.experimental.pallas import tpu as pltpu
from jax.experimental.pallas import tpu_sc as plsc

NC = 2
NS = 16
L = 16


def _aggregate(rows, cols, v, unique_nodes_list, W):
    E = v.shape[0]
    V, D = W.shape
    B = 10000
    DH = D // NC
    CH = 80
    E_TILE = E // NS
    NCHUNK = E_TILE // CH
    ZR = 40
    CR = 200
    R_TILE = 1000

    mesh = plsc.VectorSubcoreMesh(core_axis_name="c", subcore_axis_name="s")

    cp = pltpu.CompilerParams()
    if "needs_layout_passes" in pltpu.CompilerParams.__dataclass_fields__:
        cp = dataclasses.replace(cp, needs_layout_passes=False)

    @functools.partial(
        pl.kernel,
        out_type=jax.ShapeDtypeStruct((B, D), jnp.float32),
        mesh=mesh,
        compiler_params=cp,
        scratch_types=[
            pltpu.VMEM((CH,), jnp.int32),
            pltpu.VMEM((CH,), jnp.int32),
            pltpu.VMEM((CH,), jnp.float32),
            pltpu.VMEM((CH,), jnp.int32),
            pltpu.VMEM((CH, D), jnp.float32),
            pltpu.VMEM((CH, DH), jnp.float32),
            pltpu.VMEM((ZR, DH), jnp.float32),
            pltpu.VMEM_SHARED((B, DH), jnp.float32),
        ],
    )
    def run(rows_hbm, cols_hbm, v_hbm, unl_hbm, w_hbm, out_hbm,
            ci_v, ri_v, vv_v, idx_v, gbuf, cbuf, zbuf, acc):
        core = lax.axis_index("c")
        sub = lax.axis_index("s")

        @pl.loop(0, ZR)
        def _zr(i):
            @pl.loop(0, DH, step=L)
            def _zc(j):
                zbuf[i, pl.ds(j, L)] = jnp.zeros((L,), jnp.float32)

        r0 = sub * R_TILE

        @pl.when(sub < B // R_TILE)
        def _zinit():
            @pl.loop(0, R_TILE, step=ZR)
            def _zdma(k):
                pltpu.sync_copy(zbuf, acc.at[pl.ds(r0 + k, ZR)])

        plsc.subcore_barrier()

        e0 = sub * E_TILE

        def _weight_half(base_col):
            @pl.loop(0, CH, step=L)
            def _mul(k):
                vvec = vv_v[pl.ds(k, L)]
                for lane in range(L):
                    s = vvec[lane]
                    e = k + lane
                    for j in range(0, DH, L):
                        cbuf[e, pl.ds(j, L)] = (
                            gbuf[e, pl.ds(base_col + j, L)] * s)

        @pl.loop(0, NCHUNK)
        def _chunk(c):
            base = e0 + c * CH
            pltpu.sync_copy(cols_hbm.at[pl.ds(base, CH)], ci_v)
            pltpu.sync_copy(rows_hbm.at[pl.ds(base, CH)], ri_v)
            pltpu.sync_copy(v_hbm.at[pl.ds(base, CH)], vv_v)

            pltpu.sync_copy(unl_hbm.at[ci_v], idx_v)

            pltpu.sync_copy(w_hbm.at[idx_v], gbuf)

            @pl.when(core == 0)
            def _w0():
                _weight_half(0)

            @pl.when(core == 1)
            def _w1():
                _weight_half(DH)

            pltpu.sync_copy(cbuf, acc.at[ri_v], add=True)

        plsc.subcore_barrier()

        @pl.when(sub < B // R_TILE)
        def _copy_out():
            @pl.loop(0, R_TILE, step=CR)
            def _out(k):
                @pl.when(core == 0)
                def _o0():
                    pltpu.sync_copy(
                        acc.at[pl.ds(r0 + k, CR)],
                        out_hbm.at[pl.ds(r0 + k, CR), pl.ds(0, DH)])

                @pl.when(core == 1)
                def _o1():
                    pltpu.sync_copy(
                        acc.at[pl.ds(r0 + k, CR)],
                        out_hbm.at[pl.ds(r0 + k, CR), pl.ds(DH, DH)])

    return run(rows, cols, v, unique_nodes_list, W)


def kernel(nodes_real, indices, v, unique_nodes_list, num_sample, W):
    del num_sample
    assert nodes_real.shape[0] == 10000
    rows = indices[0].astype(jnp.int32)
    cols = indices[1].astype(jnp.int32)
    return _aggregate(rows, cols, v, unique_nodes_list.astype(jnp.int32), W)


# speedup vs baseline: 2.8213x; 1.8260x over previous
import dataclasses
import functools

import jax
import jax.numpy as jnp
from jax import lax
from jax.experimental import pallas as pl
from jax.experimental.pallas import tpu as pltpu
from jax.experimental.pallas import tpu_sc as plsc

NC = 2
NS = 16
L = 16
CH = 32
ET = 10240


def _aggregate(rows, cols, v, unique_nodes_list, W):
    E = v.shape[0]
    V, D = W.shape
    DH = D // NC
    B = 10000
    ET_LAST = E - (NS - 1) * ET
    ZR = 40
    CR = 200

    Wf = W.reshape(2 * V, DH)

    mesh = plsc.VectorSubcoreMesh(core_axis_name="c", subcore_axis_name="s")

    cp = pltpu.CompilerParams()
    if "needs_layout_passes" in pltpu.CompilerParams.__dataclass_fields__:
        cp = dataclasses.replace(cp, needs_layout_passes=False)

    @functools.partial(
        pl.kernel,
        out_type=jax.ShapeDtypeStruct((B, D), jnp.float32),
        mesh=mesh,
        compiler_params=cp,
        scratch_types=[
            pltpu.VMEM((ET,), jnp.int32),
            pltpu.VMEM((ET,), jnp.int32),
            pltpu.VMEM((ET,), jnp.float32),
            pltpu.VMEM((CH,), jnp.int32),
            pltpu.VMEM((CH,), jnp.int32),
            pltpu.VMEM((CH,), jnp.int32),
            pltpu.VMEM((CH,), jnp.int32),
            pltpu.VMEM((CH,), jnp.int32),
            pltpu.VMEM((CH,), jnp.int32),
            pltpu.VMEM((CH, DH), jnp.float32),
            pltpu.VMEM((CH, DH), jnp.float32),
            pltpu.VMEM((ZR, DH), jnp.float32),
            pltpu.VMEM_SHARED((B, DH), jnp.float32),
            pltpu.SemaphoreType.DMA,
            pltpu.SemaphoreType.DMA,
            pltpu.SemaphoreType.DMA,
            pltpu.SemaphoreType.DMA,
        ],
    )
    def run(rows_hbm, cols_hbm, v_hbm, unl_hbm, wf_hbm, out_hbm,
            ci_all, ri_all, vv_all, idxA, idxB, iwA, iwB, irA, irB,
            gbufA, gbufB, zbuf, acc,
            gsemA, gsemB, ssemA, ssemB):
        core = lax.axis_index("c")
        sub = lax.axis_index("s")
        e0 = sub * ET
        nchunk = jnp.where(sub == NS - 1, ET_LAST // CH, ET // CH)

        @pl.loop(0, ZR)
        def _zr(i):
            @pl.loop(0, DH, step=L)
            def _zc(j):
                zbuf[i, pl.ds(j, L)] = jnp.zeros((L,), jnp.float32)

        @pl.when(sub < B // 1000)
        def _zinit():
            @pl.loop(0, 1000, step=ZR)
            def _zdma(k):
                pltpu.sync_copy(zbuf, acc.at[pl.ds(sub * 1000 + k, ZR)])

        @pl.when(sub < NS - 1)
        def _ldmain():
            pltpu.sync_copy(cols_hbm.at[pl.ds(e0, ET)], ci_all)
            pltpu.sync_copy(rows_hbm.at[pl.ds(e0, ET)], ri_all)
            pltpu.sync_copy(v_hbm.at[pl.ds(e0, ET)], vv_all)

        @pl.when(sub == NS - 1)
        def _ldtail():
            pltpu.sync_copy(cols_hbm.at[pl.ds(e0, ET_LAST)],
                            ci_all.at[pl.ds(0, ET_LAST)])
            pltpu.sync_copy(rows_hbm.at[pl.ds(e0, ET_LAST)],
                            ri_all.at[pl.ds(0, ET_LAST)])
            pltpu.sync_copy(v_hbm.at[pl.ds(e0, ET_LAST)],
                            vv_all.at[pl.ds(0, ET_LAST)])

        plsc.subcore_barrier()

        def compose(c, idx_v, iw_v, ir_v):
            pltpu.sync_copy(unl_hbm.at[ci_all.at[pl.ds(c * CH, CH)]], idx_v)
            for g in (0, L):
                iw_v[pl.ds(g, L)] = idx_v[pl.ds(g, L)] * 2 + core
                ir_v[pl.ds(g, L)] = ri_all[pl.ds(c * CH + g, L)]

        def gather(iw_v, gbuf, gsem):
            pltpu.async_copy(wf_hbm.at[iw_v], gbuf, gsem)

        def gather_wait(iw_v, gbuf, gsem):
            pltpu.make_async_copy(wf_hbm.at[iw_v], gbuf, gsem).wait()

        def weight(c, gbuf):
            for g in (0, L):
                vvec = vv_all[pl.ds(c * CH + g, L)]
                for lane in range(L):
                    s = vvec[lane]
                    r = g + lane
                    for j in range(0, DH, L):
                        gbuf[r, pl.ds(j, L)] = gbuf[r, pl.ds(j, L)] * s

        def scat(ir_v, gbuf, ssem):
            pltpu.async_copy(gbuf, acc.at[ir_v], ssem, add=True)

        def scat_wait(ir_v, gbuf, ssem):
            pltpu.make_async_copy(gbuf, acc.at[ir_v], ssem).wait()

        compose(0, idxA, iwA, irA)
        gather(iwA, gbufA, gsemA)
        compose(1, idxB, iwB, irB)
        gather(iwB, gbufB, gsemB)

        @pl.loop(2, nchunk, step=2)
        def _body(c):
            gather_wait(iwA, gbufA, gsemA)
            weight(c - 2, gbufA)
            scat(irA, gbufA, ssemA)

            gather_wait(iwB, gbufB, gsemB)
            weight(c - 1, gbufB)
            scat(irB, gbufB, ssemB)

            scat_wait(irA, gbufA, ssemA)
            compose(c, idxA, iwA, irA)
            gather(iwA, gbufA, gsemA)

            scat_wait(irB, gbufB, ssemB)
            compose(c + 1, idxB, iwB, irB)
            gather(iwB, gbufB, gsemB)

        gather_wait(iwA, gbufA, gsemA)
        weight(nchunk - 2, gbufA)
        scat(irA, gbufA, ssemA)
        gather_wait(iwB, gbufB, gsemB)
        weight(nchunk - 1, gbufB)
        scat(irB, gbufB, ssemB)
        scat_wait(irA, gbufA, ssemA)
        scat_wait(irB, gbufB, ssemB)

        plsc.subcore_barrier()

        @pl.when(sub < B // 1000)
        def _copy_out():
            @pl.loop(0, 1000, step=CR)
            def _out(k):
                @pl.when(core == 0)
                def _o0():
                    pltpu.sync_copy(
                        acc.at[pl.ds(sub * 1000 + k, CR)],
                        out_hbm.at[pl.ds(sub * 1000 + k, CR), pl.ds(0, DH)])

                @pl.when(core == 1)
                def _o1():
                    pltpu.sync_copy(
                        acc.at[pl.ds(sub * 1000 + k, CR)],
                        out_hbm.at[pl.ds(sub * 1000 + k, CR), pl.ds(DH, DH)])

    return run(rows, cols, v, unique_nodes_list, Wf)


def kernel(nodes_real, indices, v, unique_nodes_list, num_sample, W):
    del num_sample
    assert nodes_real.shape[0] == 10000
    rows = indices[0].astype(jnp.int32)
    cols = indices[1].astype(jnp.int32)
    return _aggregate(rows, cols, v, unique_nodes_list.astype(jnp.int32), W)


# speedup vs baseline: 4.1019x; 1.4539x over previous
import dataclasses
import functools

import jax
import jax.numpy as jnp
from jax import lax
from jax.experimental import pallas as pl
from jax.experimental.pallas import tpu as pltpu
from jax.experimental.pallas import tpu_sc as plsc

NC = 2
NS = 16
L = 16
CH = 64
PB = 128
ET = 10240


def _aggregate(rows, cols, v, unique_nodes_list, W):
    E = v.shape[0]
    V, D = W.shape
    DH = D // NC
    B = 10000
    ET_LAST = E - (NS - 1) * ET
    CR = 200

    Wf = W.reshape(2 * V, DH)
    zeros = jnp.zeros((1000, DH), jnp.float32)

    mesh = plsc.VectorSubcoreMesh(core_axis_name="c", subcore_axis_name="s")

    cp = pltpu.CompilerParams()
    if "needs_layout_passes" in pltpu.CompilerParams.__dataclass_fields__:
        cp = dataclasses.replace(cp, needs_layout_passes=False)

    @functools.partial(
        pl.kernel,
        out_type=jax.ShapeDtypeStruct((B, D), jnp.float32),
        mesh=mesh,
        compiler_params=cp,
        scratch_types=[
            pltpu.VMEM((ET,), jnp.int32),
            pltpu.VMEM((ET,), jnp.int32),
            pltpu.VMEM((ET,), jnp.float32),
            pltpu.VMEM((PB,), jnp.int32),
            pltpu.VMEM((PB,), jnp.int32),
            pltpu.VMEM((CH,), jnp.int32),
            pltpu.VMEM((CH,), jnp.int32),
            pltpu.VMEM((CH, DH), jnp.float32),
            pltpu.VMEM((CH, DH), jnp.float32),
            pltpu.VMEM_SHARED((B, DH), jnp.float32),
            pltpu.SemaphoreType.DMA,
            pltpu.SemaphoreType.DMA,
            pltpu.SemaphoreType.DMA,
            pltpu.SemaphoreType.DMA,
        ],
    )
    def run(rows_hbm, cols_hbm, v_hbm, unl_hbm, wf_hbm, z_hbm, out_hbm,
            ci_all, ri_all, vv_all, bnA, bnB, irA, irB,
            gbufA, gbufB, acc,
            gsemA, gsemB, ssemA, ssemB):
        core = lax.axis_index("c")
        sub = lax.axis_index("s")
        e0 = sub * ET
        is_last = sub == NS - 1
        nchunk = jnp.where(is_last, ET_LAST // CH, ET // CH)
        nblk = jnp.where(is_last, ET_LAST // PB, ET // PB)

        @pl.when(sub < B // 1000)
        def _zinit():
            pltpu.sync_copy(z_hbm, acc.at[pl.ds(sub * 1000, 1000)])

        @pl.when(jnp.logical_not(is_last))
        def _ldmain():
            pltpu.sync_copy(cols_hbm.at[pl.ds(e0, ET)], ci_all)
            pltpu.sync_copy(rows_hbm.at[pl.ds(e0, ET)], ri_all)
            pltpu.sync_copy(v_hbm.at[pl.ds(e0, ET)], vv_all)

        @pl.when(is_last)
        def _ldtail():
            pltpu.sync_copy(cols_hbm.at[pl.ds(e0, ET_LAST)],
                            ci_all.at[pl.ds(0, ET_LAST)])
            pltpu.sync_copy(rows_hbm.at[pl.ds(e0, ET_LAST)],
                            ri_all.at[pl.ds(0, ET_LAST)])
            pltpu.sync_copy(v_hbm.at[pl.ds(e0, ET_LAST)],
                            vv_all.at[pl.ds(0, ET_LAST)])

        def eg(kb, bn, sem):
            pltpu.async_copy(
                unl_hbm.at[ci_all.at[pl.ds(kb * PB, PB)]], bn, sem)

        def eg_wait(bn, sem):
            pltpu.make_async_copy(unl_hbm.at[ci_all.at[pl.ds(0, PB)]],
                                  bn, sem).wait()

        def wb(kb, bn):
            for s in range(PB // L):
                ci_all[pl.ds(kb * PB + s * L, L)] = (
                    bn[pl.ds(s * L, L)] * 2 + core)

        eg(0, bnA, gsemA)
        eg(1, bnB, gsemB)

        @pl.loop(2, nblk, step=2)
        def _pre(kb):
            eg_wait(bnA, gsemA)
            wb(kb - 2, bnA)
            eg(kb, bnA, gsemA)
            eg_wait(bnB, gsemB)
            wb(kb - 1, bnB)
            eg(kb + 1, bnB, gsemB)

        eg_wait(bnA, gsemA)
        wb(nblk - 2, bnA)
        eg_wait(bnB, gsemB)
        wb(nblk - 1, bnB)

        plsc.subcore_barrier()

        def compose(c, ir_v):
            for g in range(0, CH, L):
                ir_v[pl.ds(g, L)] = ri_all[pl.ds(c * CH + g, L)]

        def gather(c, gbuf, gsem):
            pltpu.async_copy(
                wf_hbm.at[ci_all.at[pl.ds(c * CH, CH)]], gbuf, gsem)

        def gather_wait(gbuf, gsem):
            pltpu.make_async_copy(
                wf_hbm.at[ci_all.at[pl.ds(0, CH)]], gbuf, gsem).wait()

        def weight(c, gbuf):
            for g in range(0, CH, L):
                vvec = vv_all[pl.ds(c * CH + g, L)]
                for lane in range(L):
                    s = vvec[lane]
                    r = g + lane
                    for j in range(0, DH, L):
                        gbuf[r, pl.ds(j, L)] = gbuf[r, pl.ds(j, L)] * s

        def scat(ir_v, gbuf, ssem):
            pltpu.async_copy(gbuf, acc.at[ir_v], ssem, add=True)

        def scat_wait(ir_v, gbuf, ssem):
            pltpu.make_async_copy(gbuf, acc.at[ir_v], ssem).wait()

        compose(0, irA)
        gather(0, gbufA, gsemA)
        compose(1, irB)
        gather(1, gbufB, gsemB)

        @pl.loop(2, nchunk, step=2)
        def _body(c):
            gather_wait(gbufA, gsemA)
            weight(c - 2, gbufA)
            scat(irA, gbufA, ssemA)

            gather_wait(gbufB, gsemB)
            weight(c - 1, gbufB)
            scat(irB, gbufB, ssemB)

            scat_wait(irA, gbufA, ssemA)
            compose(c, irA)
            gather(c, gbufA, gsemA)

            scat_wait(irB, gbufB, ssemB)
            compose(c + 1, irB)
            gather(c + 1, gbufB, gsemB)

        gather_wait(gbufA, gsemA)
        weight(nchunk - 2, gbufA)
        scat(irA, gbufA, ssemA)
        gather_wait(gbufB, gsemB)
        weight(nchunk - 1, gbufB)
        scat(irB, gbufB, ssemB)
        scat_wait(irA, gbufA, ssemA)
        scat_wait(irB, gbufB, ssemB)

        plsc.subcore_barrier()

        @pl.when(sub < B // 1000)
        def _copy_out():
            @pl.loop(0, 1000, step=CR)
            def _out(k):
                @pl.when(core == 0)
                def _o0():
                    pltpu.sync_copy(
                        acc.at[pl.ds(sub * 1000 + k, CR)],
                        out_hbm.at[pl.ds(sub * 1000 + k, CR), pl.ds(0, DH)])

                @pl.when(core == 1)
                def _o1():
                    pltpu.sync_copy(
                        acc.at[pl.ds(sub * 1000 + k, CR)],
                        out_hbm.at[pl.ds(sub * 1000 + k, CR), pl.ds(DH, DH)])

    return run(rows, cols, v, unique_nodes_list, Wf, zeros)


def kernel(nodes_real, indices, v, unique_nodes_list, num_sample, W):
    del num_sample
    assert nodes_real.shape[0] == 10000
    rows = indices[0].astype(jnp.int32)
    cols = indices[1].astype(jnp.int32)
    return _aggregate(rows, cols, v, unique_nodes_list.astype(jnp.int32), W)


# speedup vs baseline: 4.3078x; 1.0502x over previous
import dataclasses
import functools

import jax
import jax.numpy as jnp
from jax import lax
from jax.experimental import pallas as pl
from jax.experimental.pallas import tpu as pltpu
from jax.experimental.pallas import tpu_sc as plsc

NC = 2
NS = 16
L = 16
CH = 32
NB = 4
PB = 128
ET = 10240


def _aggregate(rows, cols, v, unique_nodes_list, W):
    E = v.shape[0]
    V, D = W.shape
    DH = D // NC
    B = 10000
    ET_LAST = E - (NS - 1) * ET
    CR = 200

    Wf = W.reshape(2 * V, DH)
    zeros = jnp.zeros((1000, DH), jnp.float32)

    mesh = plsc.VectorSubcoreMesh(core_axis_name="c", subcore_axis_name="s")

    cp = pltpu.CompilerParams()
    if "needs_layout_passes" in pltpu.CompilerParams.__dataclass_fields__:
        cp = dataclasses.replace(cp, needs_layout_passes=False)

    @functools.partial(
        pl.kernel,
        out_type=jax.ShapeDtypeStruct((B, D), jnp.float32),
        mesh=mesh,
        compiler_params=cp,
        scratch_types=[
            pltpu.VMEM((ET,), jnp.int32),
            pltpu.VMEM((ET,), jnp.int32),
            pltpu.VMEM((ET,), jnp.float32),
            pltpu.VMEM((PB,), jnp.int32),
            pltpu.VMEM((PB,), jnp.int32),
            [pltpu.VMEM((CH,), jnp.int32) for _ in range(NB)],
            [pltpu.VMEM((CH, DH), jnp.float32) for _ in range(NB)],
            pltpu.VMEM_SHARED((B, DH), jnp.float32),
            [pltpu.SemaphoreType.DMA for _ in range(NB)],
            [pltpu.SemaphoreType.DMA for _ in range(NB)],
        ],
    )
    def run(rows_hbm, cols_hbm, v_hbm, unl_hbm, wf_hbm, z_hbm, out_hbm,
            ci_all, ri_all, vv_all, bnA, bnB, ir, gbuf, acc, gsem, ssem):
        core = lax.axis_index("c")
        sub = lax.axis_index("s")
        e0 = sub * ET
        is_last = sub == NS - 1
        nchunk = jnp.where(is_last, ET_LAST // CH, ET // CH)
        nblk = jnp.where(is_last, ET_LAST // PB, ET // PB)

        @pl.when(sub < B // 1000)
        def _zinit():
            pltpu.sync_copy(z_hbm, acc.at[pl.ds(sub * 1000, 1000)])

        @pl.when(jnp.logical_not(is_last))
        def _ldmain():
            pltpu.sync_copy(cols_hbm.at[pl.ds(e0, ET)], ci_all)
            pltpu.sync_copy(rows_hbm.at[pl.ds(e0, ET)], ri_all)
            pltpu.sync_copy(v_hbm.at[pl.ds(e0, ET)], vv_all)

        @pl.when(is_last)
        def _ldtail():
            pltpu.sync_copy(cols_hbm.at[pl.ds(e0, ET_LAST)],
                            ci_all.at[pl.ds(0, ET_LAST)])
            pltpu.sync_copy(rows_hbm.at[pl.ds(e0, ET_LAST)],
                            ri_all.at[pl.ds(0, ET_LAST)])
            pltpu.sync_copy(v_hbm.at[pl.ds(e0, ET_LAST)],
                            vv_all.at[pl.ds(0, ET_LAST)])

        def eg(kb, bn, sem):
            pltpu.async_copy(
                unl_hbm.at[ci_all.at[pl.ds(kb * PB, PB)]], bn, sem)

        def eg_wait(bn, sem):
            pltpu.make_async_copy(unl_hbm.at[ci_all.at[pl.ds(0, PB)]],
                                  bn, sem).wait()

        def wb(kb, bn):
            for s in range(PB // L):
                ci_all[pl.ds(kb * PB + s * L, L)] = (
                    bn[pl.ds(s * L, L)] * 2 + core)

        eg(0, bnA, gsem[0])
        eg(1, bnB, gsem[1])

        @pl.loop(2, nblk, step=2)
        def _pre(kb):
            eg_wait(bnA, gsem[0])
            wb(kb - 2, bnA)
            eg(kb, bnA, gsem[0])
            eg_wait(bnB, gsem[1])
            wb(kb - 1, bnB)
            eg(kb + 1, bnB, gsem[1])

        eg_wait(bnA, gsem[0])
        wb(nblk - 2, bnA)
        eg_wait(bnB, gsem[1])
        wb(nblk - 1, bnB)

        plsc.subcore_barrier()

        def compose(c, q):
            for g in range(0, CH, L):
                ir[q][pl.ds(g, L)] = ri_all[pl.ds(c * CH + g, L)]

        def gather(c, q):
            pltpu.async_copy(
                wf_hbm.at[ci_all.at[pl.ds(c * CH, CH)]], gbuf[q], gsem[q])

        def gather_wait(q):
            pltpu.make_async_copy(
                wf_hbm.at[ci_all.at[pl.ds(0, CH)]], gbuf[q], gsem[q]).wait()

        def weight(c, q):
            for g in range(0, CH, L):
                vvec = vv_all[pl.ds(c * CH + g, L)]
                for lane in range(L):
                    s = vvec[lane]
                    r = g + lane
                    for j in range(0, DH, L):
                        gbuf[q][r, pl.ds(j, L)] = gbuf[q][r, pl.ds(j, L)] * s

        def scat(q):
            pltpu.async_copy(gbuf[q], acc.at[ir[q]], ssem[q], add=True)

        def scat_wait(q):
            pltpu.make_async_copy(gbuf[q], acc.at[ir[q]], ssem[q]).wait()

        for q in range(NB):
            compose(q, q)
            gather(q, q)

        @pl.loop(NB, nchunk, step=NB)
        def _body(c):
            for q in range(NB):
                gather_wait(q)
                weight(c - NB + q, q)
                scat(q)

            for q in range(NB):
                scat_wait(q)
                compose(c + q, q)
                gather(c + q, q)

        for q in range(NB):
            gather_wait(q)
            weight(nchunk - NB + q, q)
            scat(q)
        for q in range(NB):
            scat_wait(q)

        plsc.subcore_barrier()

        @pl.when(sub < B // 1000)
        def _copy_out():
            @pl.loop(0, 1000, step=CR)
            def _out(k):
                @pl.when(core == 0)
                def _o0():
                    pltpu.sync_copy(
                        acc.at[pl.ds(sub * 1000 + k, CR)],
                        out_hbm.at[pl.ds(sub * 1000 + k, CR), pl.ds(0, DH)])

                @pl.when(core == 1)
                def _o1():
                    pltpu.sync_copy(
                        acc.at[pl.ds(sub * 1000 + k, CR)],
                        out_hbm.at[pl.ds(sub * 1000 + k, CR), pl.ds(DH, DH)])

    return run(rows, cols, v, unique_nodes_list, Wf, zeros)


def kernel(nodes_real, indices, v, unique_nodes_list, num_sample, W):
    del num_sample
    assert nodes_real.shape[0] == 10000
    rows = indices[0].astype(jnp.int32)
    cols = indices[1].astype(jnp.int32)
    return _aggregate(rows, cols, v, unique_nodes_list.astype(jnp.int32), W)
